# baseline (speedup 1.0000x reference)
import jax
import jax.numpy as jnp
from jax import lax
from jax.experimental import pallas as pl
from jax.experimental.pallas import tpu as pltpu
from jax.experimental.pallas import tpu_sc as plsc

BATCH = 4096
SEQ_LEN = 200
EMBED_DIM = 64
VOCAB = 1000000
ROW_PAD = 128

NUM_CORES = 2
NUM_SUBCORES = 16
NUM_WORKERS = NUM_CORES * NUM_SUBCORES

BLOCK_B = BATCH // NUM_WORKERS
NBUF = 3
LANES = 16
GROUPS = EMBED_DIM // LANES


def _wid():
    return lax.axis_index("s") * NUM_CORES + lax.axis_index("c")


def _embed_body(idx_hbm, table_hbm, pos_hbm, out_hbm, idx_v, rows_bufs,
                out_bufs, pos_v, gsems, wsems):
    wid = _wid()
    b0 = wid * BLOCK_B

    pltpu.sync_copy(pos_hbm, pos_v)
    pltpu.sync_copy(idx_hbm.at[:, pl.ds(b0, BLOCK_B)], idx_v)

    iota = lax.iota(jnp.int32, LANES)
    rot = [jnp.bitwise_and(iota + k, LANES - 1) for k in range(LANES)]
    ge = [iota + be for be in range(0, EMBED_DIM, LANES)]

    def gather_copy(b, l):
        return pltpu.make_async_copy(
            table_hbm.at[idx_v.at[l]], rows_bufs[b], gsems[b])

    def write_copy(b, l):
        return pltpu.make_async_copy(
            out_bufs[b], out_hbm.at[l, :, pl.ds(b0, BLOCK_B)], wsems[b])

    def transpose_add(b, l):
        pos_g = [pos_v[l, pl.ds(g * LANES, LANES)] for g in range(GROUPS)]

        def tbody(bt8, carry):
            bbv = jnp.broadcast_to(bt8 * LANES, (LANES,)).astype(jnp.int32)
            for k in range(LANES):
                bcol = bbv + rot[k]
                for g in range(GROUPS):
                    v = plsc.load_gather(rows_bufs[b], [bcol, ge[g]])
                    plsc.store_scatter(out_bufs[b], [ge[g], bcol],
                                       v + pos_g[g])
            return carry

        lax.fori_loop(0, BLOCK_B // LANES, tbody, 0, unroll=4)

    for b in range(NBUF - 1):
        gather_copy(b, b).start()

    def outer(o, carry):
        for b in range(NBUF):
            l = o * NBUF + b
            gather_copy(b, l).wait()

            @pl.when(l >= NBUF)
            def _():
                write_copy(b, l - NBUF).wait()

            transpose_add(b, l)

            @pl.when(l + NBUF - 1 <= SEQ_LEN - 1)
            def _():
                gather_copy((b - 1) % NBUF, l + NBUF - 1).start()

            write_copy(b, l).start()
        return carry

    lax.fori_loop(0, SEQ_LEN // NBUF, outer, 0, unroll=False)

    for l in range(SEQ_LEN - SEQ_LEN % NBUF, SEQ_LEN):
        b = l % NBUF
        gather_copy(b, l).wait()
        write_copy(b, l - NBUF).wait()
        transpose_add(b, l)
        write_copy(b, l).start()

    for l in range(SEQ_LEN - NBUF, SEQ_LEN):
        write_copy(l % NBUF, l).wait()


@jax.jit
def _embed(inputs, token_table, pos_table):
    mesh = plsc.VectorSubcoreMesh(
        core_axis_name="c", subcore_axis_name="s", num_cores=NUM_CORES,
        num_subcores=NUM_SUBCORES)

    table_p = jnp.pad(token_table, ((0, 0), (0, ROW_PAD - EMBED_DIM)))
    idx_t = jnp.transpose(inputs)

    f = pl.kernel(
        _embed_body,
        out_type=jax.ShapeDtypeStruct((SEQ_LEN, EMBED_DIM, BATCH),
                                      jnp.float32),
        mesh=mesh,
        scratch_types=[
            pltpu.VMEM((SEQ_LEN, BLOCK_B), jnp.int32),
            [pltpu.VMEM((BLOCK_B, ROW_PAD), jnp.float32)] * NBUF,
            [pltpu.VMEM((EMBED_DIM, BLOCK_B), jnp.float32)] * NBUF,
            pltpu.VMEM((SEQ_LEN, EMBED_DIM), jnp.float32),
            [pltpu.SemaphoreType.DMA] * NBUF,
            [pltpu.SemaphoreType.DMA] * NBUF,
        ],
        compiler_params=pltpu.CompilerParams(use_tc_tiling_on_sc=True,
                                             needs_layout_passes=False),
    )
    out_t = f(idx_t, table_p, pos_table)
    return jnp.transpose(out_t, (2, 0, 1))


def kernel(inputs, token_table, pos_table):
    return _embed(inputs, token_table, pos_table)
